# Initial kernel scaffold; baseline (speedup 1.0000x reference)
import jax
import jax.numpy as jnp
from jax.experimental import pallas as pl


def kernel(x, Wq, bq, Wk, bk, Wv, bv):
    raise NotImplementedError("write your pallas kernel here")


# speedup vs baseline: 5.7165x; 5.7165x over previous
import functools

import jax
import jax.numpy as jnp
from jax.experimental import pallas as pl

TOPK = 8
NEG_INF = float("-inf")


def _proj_body(x_ref, w_ref, b_ref, o_ref):
    acc = jnp.dot(x_ref[...], w_ref[...], preferred_element_type=jnp.float32)
    o_ref[...] = (acc + b_ref[...].astype(jnp.float32)).astype(jnp.bfloat16)


def _attn_body(q_ref, k_ref, v_ref, o_ref):
    q = q_ref[0]
    k = k_ref[0]
    scores = jax.lax.dot_general(
        q, k, (((1,), (1,)), ((), ())), preferred_element_type=jnp.float32
    )
    cur = scores
    total = jnp.zeros((scores.shape[0], 1), jnp.float32)
    for _ in range(TOPK):
        m = jnp.max(cur, axis=1, keepdims=True)
        total = total + m
        cur = jnp.where(cur == m, NEG_INF, cur)
    w = jnp.where(cur == NEG_INF, scores, 0.0) / (total + 1e-10)
    o_ref[0] = jax.lax.dot_general(
        w.astype(jnp.bfloat16), v_ref[0],
        (((1,), (0,)), ((), ())), preferred_element_type=jnp.float32,
    )


def kernel(x, Wq, bq, Wk, bk, Wv, bv):
    B, S, E = x.shape
    MBLK = 512
    SBLK = 256

    x2d = x.reshape(B * S, E).astype(jnp.bfloat16)
    w_cat = jnp.concatenate([Wq.T, Wk.T, Wv.T], axis=1).astype(jnp.bfloat16)
    b_cat = jnp.concatenate([bq, bk, bv]).reshape(1, 3 * E)

    qkv = pl.pallas_call(
        _proj_body,
        grid=(3, B * S // MBLK),
        in_specs=[
            pl.BlockSpec((MBLK, E), lambda j, i: (i, 0)),
            pl.BlockSpec((E, E), lambda j, i: (0, j)),
            pl.BlockSpec((1, E), lambda j, i: (0, j)),
        ],
        out_specs=pl.BlockSpec((MBLK, E), lambda j, i: (i, j)),
        out_shape=jax.ShapeDtypeStruct((B * S, 3 * E), jnp.bfloat16),
    )(x2d, w_cat, b_cat)

    q = qkv[:, :E].reshape(B, S, E)
    kk = qkv[:, E:2 * E].reshape(B, S, E)
    vv = qkv[:, 2 * E:].reshape(B, S, E)

    out = pl.pallas_call(
        _attn_body,
        grid=(B, S // SBLK),
        in_specs=[
            pl.BlockSpec((1, SBLK, E), lambda b, i: (b, i, 0)),
            pl.BlockSpec((1, S, E), lambda b, i: (b, 0, 0)),
            pl.BlockSpec((1, S, E), lambda b, i: (b, 0, 0)),
        ],
        out_specs=pl.BlockSpec((1, SBLK, E), lambda b, i: (b, i, 0)),
        out_shape=jax.ShapeDtypeStruct((B, S, E), jnp.float32),
    )(q, kk, vv)
    return out


# speedup vs baseline: 6.3294x; 1.1072x over previous
import jax
import jax.numpy as jnp
from jax.experimental import pallas as pl

TOPK = 8
NEG_INF = float("-inf")


def _proj_body(x_ref, w_ref, b_ref, o_ref):
    acc = jnp.dot(x_ref[0], w_ref[...], preferred_element_type=jnp.float32)
    o_ref[0] = (acc + b_ref[...].astype(jnp.float32)).astype(jnp.bfloat16)


def _attn_body(q_ref, k_ref, v_ref, o_ref):
    q = q_ref[0]
    k = k_ref[0]
    scores = jax.lax.dot_general(
        q, k, (((1,), (1,)), ((), ())), preferred_element_type=jnp.float32
    )
    cur = scores
    total = jnp.zeros((scores.shape[0], 1), jnp.float32)
    for _ in range(TOPK):
        m = jnp.max(cur, axis=1, keepdims=True)
        total = total + m
        cur = jnp.where(cur == m, NEG_INF, cur)
    w = jnp.where(cur == NEG_INF, scores, 0.0) / (total + 1e-10)
    o_ref[0] = jax.lax.dot_general(
        w.astype(jnp.bfloat16), v_ref[0],
        (((1,), (0,)), ((), ())), preferred_element_type=jnp.float32,
    )


def kernel(x, Wq, bq, Wk, bk, Wv, bv):
    B, S, E = x.shape
    MBLK = 512
    SBLK = 256
    nm = S // MBLK

    x3 = x.astype(jnp.bfloat16)
    w_cat = jnp.concatenate([Wq.T, Wk.T, Wv.T], axis=1).astype(jnp.bfloat16)
    b_cat = jnp.concatenate([bq, bk, bv]).reshape(1, 3 * E)

    qkv = pl.pallas_call(
        _proj_body,
        grid=(3, B * nm),
        in_specs=[
            pl.BlockSpec((1, MBLK, E), lambda j, i: (i // nm, i % nm, 0)),
            pl.BlockSpec((E, E), lambda j, i: (0, j)),
            pl.BlockSpec((1, E), lambda j, i: (0, j)),
        ],
        out_specs=pl.BlockSpec((1, MBLK, E), lambda j, i: (i // nm, i % nm, j)),
        out_shape=jax.ShapeDtypeStruct((B, S, 3 * E), jnp.bfloat16),
    )(x3, w_cat, b_cat)

    out = pl.pallas_call(
        _attn_body,
        grid=(B, S // SBLK),
        in_specs=[
            pl.BlockSpec((1, SBLK, E), lambda b, i: (b, i, 0)),
            pl.BlockSpec((1, S, E), lambda b, i: (b, 0, 1)),
            pl.BlockSpec((1, S, E), lambda b, i: (b, 0, 2)),
        ],
        out_specs=pl.BlockSpec((1, SBLK, E), lambda b, i: (b, i, 0)),
        out_shape=jax.ShapeDtypeStruct((B, S, E), jnp.float32),
    )(qkv, qkv, qkv)
    return out


# speedup vs baseline: 6.4854x; 1.0247x over previous
import jax
import jax.numpy as jnp
from jax.experimental import pallas as pl

TOPK = 8
NEG_INF = float("-inf")


def _proj_body(x_ref, w_ref, b_ref, o_ref):
    acc = jnp.dot(x_ref[0], w_ref[...], preferred_element_type=jnp.float32)
    o_ref[0] = (acc + b_ref[...].astype(jnp.float32)).astype(jnp.bfloat16)


def _attn_body(q_ref, k_ref, v_ref, o_ref):
    q = q_ref[0]
    k = k_ref[0]
    scores = jax.lax.dot_general(
        q, k, (((1,), (1,)), ((), ())), preferred_element_type=jnp.float32
    )
    m = jnp.max(scores, axis=1, keepdims=True)
    total = m
    for _ in range(TOPK - 1):
        m = jnp.max(jnp.where(scores < m, scores, NEG_INF), axis=1, keepdims=True)
        total = total + m
    w = jnp.where(scores >= m, scores, 0.0) / (total + 1e-10)
    o_ref[0] = jax.lax.dot_general(
        w.astype(jnp.bfloat16), v_ref[0],
        (((1,), (0,)), ((), ())), preferred_element_type=jnp.float32,
    )


def kernel(x, Wq, bq, Wk, bk, Wv, bv):
    B, S, E = x.shape
    MBLK = min(1024, S)
    SBLK = min(512, S)
    nm = S // MBLK

    x3 = x.astype(jnp.bfloat16)
    w_cat = jnp.concatenate([Wq.T, Wk.T, Wv.T], axis=1).astype(jnp.bfloat16)
    b_cat = jnp.concatenate([bq, bk, bv]).reshape(1, 3 * E)

    qkv = pl.pallas_call(
        _proj_body,
        grid=(3, B * nm),
        in_specs=[
            pl.BlockSpec((1, MBLK, E), lambda j, i: (i // nm, i % nm, 0)),
            pl.BlockSpec((E, E), lambda j, i: (0, j)),
            pl.BlockSpec((1, E), lambda j, i: (0, j)),
        ],
        out_specs=pl.BlockSpec((1, MBLK, E), lambda j, i: (i // nm, i % nm, j)),
        out_shape=jax.ShapeDtypeStruct((B, S, 3 * E), jnp.bfloat16),
    )(x3, w_cat, b_cat)

    out = pl.pallas_call(
        _attn_body,
        grid=(B, S // SBLK),
        in_specs=[
            pl.BlockSpec((1, SBLK, E), lambda b, i: (b, i, 0)),
            pl.BlockSpec((1, S, E), lambda b, i: (b, 0, 1)),
            pl.BlockSpec((1, S, E), lambda b, i: (b, 0, 2)),
        ],
        out_specs=pl.BlockSpec((1, SBLK, E), lambda b, i: (b, i, 0)),
        out_shape=jax.ShapeDtypeStruct((B, S, E), jnp.float32),
    )(qkv, qkv, qkv)
    return out


# speedup vs baseline: 7.7779x; 1.1993x over previous
import jax
import jax.numpy as jnp
from jax.experimental import pallas as pl

TOPK = 8
NEG_INF = float("-inf")

_DN_T = (((1,), (1,)), ((), ()))
_DN = (((1,), (0,)), ((), ()))


def _proj_body(x_ref, wq_ref, wk_ref, wv_ref, bq_ref, bk_ref, bv_ref,
               q_ref, k_ref, v_ref):
    xb = x_ref[0].astype(jnp.bfloat16)
    for w_ref, b_ref, o_ref in ((wq_ref, bq_ref, q_ref),
                                (wk_ref, bk_ref, k_ref),
                                (wv_ref, bv_ref, v_ref)):
        acc = jax.lax.dot_general(xb, w_ref[...], _DN_T,
                                  preferred_element_type=jnp.float32)
        o_ref[0] = (acc + b_ref[...]).astype(jnp.bfloat16)


def _attn_body(q_ref, k_ref, v_ref, o_ref):
    q = q_ref[0]
    k = k_ref[0]
    scores = jax.lax.dot_general(q, k, _DN_T,
                                 preferred_element_type=jnp.float32)
    m = jnp.max(scores, axis=1, keepdims=True)
    total = m
    for _ in range(TOPK - 1):
        m = jnp.max(jnp.where(scores < m, scores, NEG_INF), axis=1,
                    keepdims=True)
        total = total + m
    w = jnp.where(scores >= m, scores, 0.0) / (total + 1e-10)
    o_ref[0] = jax.lax.dot_general(w.astype(jnp.bfloat16), v_ref[0], _DN,
                                   preferred_element_type=jnp.float32)


def kernel(x, Wq, bq, Wk, bk, Wv, bv):
    B, S, E = x.shape
    MBLK = min(512, S)
    SBLK = min(512, S)
    nm = S // MBLK

    wq = Wq.astype(jnp.bfloat16)
    wk = Wk.astype(jnp.bfloat16)
    wv = Wv.astype(jnp.bfloat16)
    b2 = lambda b: b.reshape(1, E)

    w_spec = pl.BlockSpec((E, E), lambda i: (0, 0))
    b_spec = pl.BlockSpec((1, E), lambda i: (0, 0))
    row_spec = pl.BlockSpec((1, MBLK, E), lambda i: (i // nm, i % nm, 0))
    out_sd = jax.ShapeDtypeStruct((B, S, E), jnp.bfloat16)

    q, kk, vv = pl.pallas_call(
        _proj_body,
        grid=(B * nm,),
        in_specs=[row_spec, w_spec, w_spec, w_spec, b_spec, b_spec, b_spec],
        out_specs=[row_spec, row_spec, row_spec],
        out_shape=[out_sd, out_sd, out_sd],
    )(x, wq, wk, wv, b2(bq), b2(bk), b2(bv))

    out = pl.pallas_call(
        _attn_body,
        grid=(B, S // SBLK),
        in_specs=[
            pl.BlockSpec((1, SBLK, E), lambda b, i: (b, i, 0)),
            pl.BlockSpec((1, S, E), lambda b, i: (b, 0, 0)),
            pl.BlockSpec((1, S, E), lambda b, i: (b, 0, 0)),
        ],
        out_specs=pl.BlockSpec((1, SBLK, E), lambda b, i: (b, i, 0)),
        out_shape=jax.ShapeDtypeStruct((B, S, E), jnp.float32),
    )(q, kk, vv)
    return out
